# baseline (speedup 1.0000x reference)
import functools

import jax
import jax.numpy as jnp
from jax import lax
from jax.experimental import pallas as pl
from jax.experimental.pallas import tpu as pltpu
from jax.experimental.pallas import tpu_sc as plsc

N = 10000
D = 128
E = 320000
MF = 16
NP = 10240
RPT = NP // 16
C = 64
NSC = 2
EPS = E // 32
NCH = 157
EPP = NCH * C - EPS
NBUF = 3
NCHP = 160

_mesh = plsc.VectorSubcoreMesh(core_axis_name="c", subcore_axis_name="s")


def _sc_body(with_deg, y_hbm, pk_hbm, *rest):
    if with_deg:
        (agg_hbm, deg_hbm, packb, r0, r1, r2, sv0, sv1, sv2,
         dv0, dv1, dv2, zdeg_v, ones_v,
         acc_sh, deg_sh, gsem, ssem, dsem) = rest
    else:
        (agg_hbm, packb, r0, r1, r2, sv0, sv1, sv2,
         dv0, dv1, dv2, acc_sh, gsem, ssem) = rest
    rows = (r0, r1, r2)
    srcv = (sv0, sv1, sv2)
    dstv = (dv0, dv1, dv2)

    c = lax.axis_index("c")
    s = lax.axis_index("s")

    zero16 = jnp.zeros((16,), jnp.float32)
    one16 = jnp.ones((16,), jnp.float32)

    for i in range(C):
        for k in range(D // 16):
            r0[i, pl.ds(k * 16, 16)] = zero16
    if with_deg:
        for i in range(C // 16):
            zdeg_v[pl.ds(i * 16, 16)] = zero16
            ones_v[pl.ds(i * 16, 16)] = one16

    pltpu.sync_copy(pk_hbm.at[pl.ds((c * 16 + s) * NCHP, NCHP)], packb)

    row0 = s * RPT
    for k in range(RPT // C):
        pltpu.sync_copy(r0, acc_sh.at[pl.ds(row0 + k * C, C)])
        if with_deg:
            pltpu.sync_copy(zdeg_v, deg_sh.at[pl.ds(row0 + k * C, C)])
    plsc.subcore_barrier()

    def decode(g, b):
        for k in range(C // 16):
            v16 = packb[g, pl.ds(k * 16, 16)]
            srcv[b][pl.ds(k * 16, 16)] = lax.shift_right_logical(v16, 14)
            dstv[b][pl.ds(k * 16, 16)] = lax.bitwise_and(v16, 16383)

    def gather_start(g, b):
        decode(g, b)
        pltpu.async_copy(y_hbm.at[srcv[b]], rows[b], gsem.at[b])

    def gather_wait(b):
        pltpu.make_async_copy(y_hbm.at[srcv[b]], rows[b], gsem.at[b]).wait()

    def scatter_start(b):
        pltpu.async_copy(rows[b], acc_sh.at[dstv[b]], ssem.at[b], add=True)
        if with_deg:
            pltpu.async_copy(ones_v, deg_sh.at[dstv[b]], dsem.at[b], add=True)

    def scatter_wait(b):
        pltpu.make_async_copy(rows[b], acc_sh.at[dstv[b]], ssem.at[b]).wait()
        if with_deg:
            pltpu.make_async_copy(ones_v, deg_sh.at[dstv[b]], dsem.at[b]).wait()

    gather_start(0, 0)
    gather_wait(0)
    scatter_start(0)
    gather_start(1, 1)

    def group(grp, carry):
        for j in range(NBUF):
            g = 1 + grp * NBUF + j
            b = (1 + j) % NBUF
            bm = j
            bn = (2 + j) % NBUF
            gather_wait(b)
            scatter_start(b)
            scatter_wait(bm)
            gn = jnp.minimum(g + 1, NCH - 1)
            gather_start(gn, bn)
        return carry

    lax.fori_loop(0, (NCH - 1) // NBUF, group, 0)
    scatter_wait(0)
    gather_wait(1)
    plsc.subcore_barrier()

    pltpu.sync_copy(acc_sh.at[pl.ds(row0, RPT)], agg_hbm.at[c, pl.ds(row0, RPT)])
    if with_deg:
        pltpu.sync_copy(deg_sh.at[pl.ds(row0, RPT)], deg_hbm.at[c, pl.ds(row0, RPT)])


_ROWB = [pltpu.VMEM((C, D), jnp.float32)] * NBUF
_IDXB = [pltpu.VMEM((C,), jnp.int32)] * (2 * NBUF)

_sc_agg_deg = pl.kernel(
    functools.partial(_sc_body, True),
    out_type=[
        jax.ShapeDtypeStruct((NSC, NP, D), jnp.float32),
        jax.ShapeDtypeStruct((NSC, NP), jnp.float32),
    ],
    mesh=_mesh,
    scratch_types=[
        pltpu.VMEM((NCHP, C), jnp.int32),
        *_ROWB,
        *_IDXB,
        pltpu.VMEM((C,), jnp.float32),
        pltpu.VMEM((C,), jnp.float32),
        pltpu.VMEM_SHARED((NP, D), jnp.float32),
        pltpu.VMEM_SHARED((NP,), jnp.float32),
        pltpu.SemaphoreType.DMA((NBUF,)),
        pltpu.SemaphoreType.DMA((NBUF,)),
        pltpu.SemaphoreType.DMA((NBUF,)),
    ],
)

_sc_agg = pl.kernel(
    functools.partial(_sc_body, False),
    out_type=jax.ShapeDtypeStruct((NSC, NP, D), jnp.float32),
    mesh=_mesh,
    scratch_types=[
        pltpu.VMEM((NCHP, C), jnp.int32),
        *_ROWB,
        *_IDXB,
        pltpu.VMEM_SHARED((NP, D), jnp.float32),
        pltpu.SemaphoreType.DMA((NBUF,)),
        pltpu.SemaphoreType.DMA((NBUF,)),
    ],
)


BM = 2000


def _tc_in_body(x_ref, wn_ref, ws_ref, b_ref, y_ref, z_ref):
    xb = x_ref[...]
    y_ref[...] = jnp.dot(xb, wn_ref[...], preferred_element_type=jnp.float32)
    z_ref[...] = jnp.dot(xb, ws_ref[...], preferred_element_type=jnp.float32) + b_ref[...]


_tc_in = pl.pallas_call(
    _tc_in_body,
    grid=(N // BM,),
    in_specs=[
        pl.BlockSpec((BM, D), lambda i: (i, 0)),
        pl.BlockSpec((D, D), lambda i: (0, 0)),
        pl.BlockSpec((D, D), lambda i: (0, 0)),
        pl.BlockSpec((1, D), lambda i: (0, 0)),
    ],
    out_specs=[
        pl.BlockSpec((BM, D), lambda i: (i, 0)),
        pl.BlockSpec((BM, D), lambda i: (i, 0)),
    ],
    out_shape=[
        jax.ShapeDtypeStruct((N, D), jnp.float32),
        jax.ShapeDtypeStruct((N, D), jnp.float32),
    ],
)


def _h_from_parts(z, aA, aB, dA, dB):
    deg = jnp.maximum(dA + dB, 1.0)
    h = z + (aA + aB) / deg
    return jnp.where(h >= 0, h, 0.01 * h)


_AGG_SPECS = [
    pl.BlockSpec((1, BM, D), lambda i: (0, i, 0)),
    pl.BlockSpec((1, BM, D), lambda i: (1, i, 0)),
    pl.BlockSpec((1, BM, 1), lambda i: (0, i, 0)),
    pl.BlockSpec((1, BM, 1), lambda i: (1, i, 0)),
]


def _tc_mid_body(z_ref, aA_ref, aB_ref, dA_ref, dB_ref, wn_ref, ws_ref, b_ref,
                 y_ref, z2_ref):
    h = _h_from_parts(z_ref[...], aA_ref[0], aB_ref[0], dA_ref[0], dB_ref[0])
    y_ref[...] = jnp.dot(h, wn_ref[...], preferred_element_type=jnp.float32)
    z2_ref[...] = jnp.dot(h, ws_ref[...], preferred_element_type=jnp.float32) + b_ref[...]


_tc_mid = pl.pallas_call(
    _tc_mid_body,
    grid=(N // BM,),
    in_specs=[pl.BlockSpec((BM, D), lambda i: (i, 0))] + _AGG_SPECS + [
        pl.BlockSpec((D, D), lambda i: (0, 0)),
        pl.BlockSpec((D, D), lambda i: (0, 0)),
        pl.BlockSpec((1, D), lambda i: (0, 0)),
    ],
    out_specs=[
        pl.BlockSpec((BM, D), lambda i: (i, 0)),
        pl.BlockSpec((BM, D), lambda i: (i, 0)),
    ],
    out_shape=[
        jax.ShapeDtypeStruct((N, D), jnp.float32),
        jax.ShapeDtypeStruct((N, D), jnp.float32),
    ],
)


def _tc_fin_body(z_ref, aA_ref, aB_ref, dA_ref, dB_ref, mf_ref, w3a_ref,
                 w3b_ref, b3_ref, w4_ref, b4_ref, sum_ref, o_ref):
    h = _h_from_parts(z_ref[...], aA_ref[0], aB_ref[0], dA_ref[0], dB_ref[0])
    part = jnp.sum(h, axis=0, keepdims=True)

    @pl.when(pl.program_id(0) == 0)
    def _init():
        sum_ref[...] = part

    @pl.when(pl.program_id(0) != 0)
    def _acc():
        sum_ref[...] += part

    @pl.when(pl.program_id(0) == pl.num_programs(0) - 1)
    def _head():
        g = sum_ref[...] * (1.0 / N)
        t = (jnp.dot(g, w3a_ref[...], preferred_element_type=jnp.float32)
             + jnp.dot(mf_ref[...], w3b_ref[...], preferred_element_type=jnp.float32)
             + b3_ref[...])
        a = jnp.maximum(t, 0.0)
        o_ref[...] = jnp.dot(a, w4_ref[...], preferred_element_type=jnp.float32) + b4_ref[...]


_tc_fin = pl.pallas_call(
    _tc_fin_body,
    grid=(N // BM,),
    in_specs=[pl.BlockSpec((BM, D), lambda i: (i, 0))] + _AGG_SPECS + [
        pl.BlockSpec((1, MF), lambda i: (0, 0)),
        pl.BlockSpec((D, 64), lambda i: (0, 0)),
        pl.BlockSpec((MF, 64), lambda i: (0, 0)),
        pl.BlockSpec((1, 64), lambda i: (0, 0)),
        pl.BlockSpec((64, 1), lambda i: (0, 0)),
        pl.BlockSpec((1, 1), lambda i: (0, 0)),
    ],
    out_specs=[
        pl.BlockSpec((1, D), lambda i: (0, 0)),
        pl.BlockSpec((1, 1), lambda i: (0, 0)),
    ],
    out_shape=[
        jax.ShapeDtypeStruct((1, D), jnp.float32),
        jax.ShapeDtypeStruct((1, 1), jnp.float32),
    ],
)


def kernel(x, edge_index, manual_features, W1_self, W1_neigh, b1,
           W2_self, W2_neigh, b2, W3, b3, W4, b4):
    packed = (edge_index[0] * 16384 + edge_index[1]).reshape(32, EPS)
    padv = (N + (jnp.arange(32 * EPP, dtype=jnp.int32) % (NP - N))).reshape(32, EPP)
    pk2 = jnp.pad(jnp.concatenate([packed, padv], axis=1).reshape(32, NCH, C),
                  ((0, 0), (0, NCHP - NCH), (0, 0))).reshape(32 * NCHP, C)

    y1, z1 = _tc_in(x, W1_neigh, W1_self, b1.reshape(1, D))
    agg1, deg = _sc_agg_deg(y1, pk2)
    deg3 = deg.reshape(NSC, NP, 1)

    y2, z2 = _tc_mid(z1, agg1, agg1, deg3, deg3,
                     W2_neigh, W2_self, b2.reshape(1, D))
    agg2 = _sc_agg(y2, pk2)

    _, res = _tc_fin(z2, agg2, agg2, deg3, deg3,
                     manual_features.reshape(1, MF),
                     W3[:D], W3[D:], b3.reshape(1, -1), W4, b4.reshape(1, 1))
    return res.reshape((1,))


# speedup vs baseline: 1.0201x; 1.0201x over previous
import functools

import jax
import jax.numpy as jnp
from jax import lax
from jax.experimental import pallas as pl
from jax.experimental.pallas import tpu as pltpu
from jax.experimental.pallas import tpu_sc as plsc

N = 10000
D = 128
E = 320000
MF = 16
NP = 10240
RPT = NP // 16
C = 64
NSC = 2
EPS = E // 32
NCH = 157
EPP = NCH * C - EPS
NBUF = 3
NCHP = 160

_mesh = plsc.VectorSubcoreMesh(core_axis_name="c", subcore_axis_name="s")


def _sc_body(with_deg, y_hbm, pk_hbm, *rest):
    if with_deg:
        (agg_hbm, deg_hbm, packb, r0, r1, r2, sv0, sv1, sv2,
         dv0, dv1, dv2, zdeg_v, ones_v,
         acc_sh, deg_sh, gsem, ssem, dsem) = rest
    else:
        (agg_hbm, packb, r0, r1, r2, sv0, sv1, sv2,
         dv0, dv1, dv2, acc_sh, gsem, ssem) = rest
    rows = (r0, r1, r2)
    srcv = (sv0, sv1, sv2)
    dstv = (dv0, dv1, dv2)

    c = lax.axis_index("c")
    s = lax.axis_index("s")

    zero16 = jnp.zeros((16,), jnp.float32)
    one16 = jnp.ones((16,), jnp.float32)

    for i in range(C):
        for k in range(D // 16):
            r0[i, pl.ds(k * 16, 16)] = zero16
    if with_deg:
        for i in range(C // 16):
            zdeg_v[pl.ds(i * 16, 16)] = zero16
            ones_v[pl.ds(i * 16, 16)] = one16

    pltpu.sync_copy(pk_hbm.at[pl.ds((c * 16 + s) * NCHP, NCHP)], packb)

    row0 = s * RPT
    for k in range(RPT // C):
        pltpu.sync_copy(r0, acc_sh.at[pl.ds(row0 + k * C, C)])
        if with_deg:
            pltpu.sync_copy(zdeg_v, deg_sh.at[pl.ds(row0 + k * C, C)])
    plsc.subcore_barrier()

    def decode(g, b):
        for k in range(C // 16):
            v16 = packb[g, pl.ds(k * 16, 16)]
            srcv[b][pl.ds(k * 16, 16)] = lax.shift_right_logical(v16, 14)
            dstv[b][pl.ds(k * 16, 16)] = lax.bitwise_and(v16, 16383)

    def gather_start(g, b):
        decode(g, b)
        pltpu.async_copy(y_hbm.at[srcv[b]], rows[b], gsem.at[b])

    def gather_wait(b):
        pltpu.make_async_copy(y_hbm.at[srcv[b]], rows[b], gsem.at[b]).wait()

    def scatter_start(b):
        pltpu.async_copy(rows[b], acc_sh.at[dstv[b]], ssem.at[b], add=True)
        if with_deg:
            pltpu.async_copy(ones_v, deg_sh.at[dstv[b]], dsem.at[b], add=True)

    def scatter_wait(b):
        pltpu.make_async_copy(rows[b], acc_sh.at[dstv[b]], ssem.at[b]).wait()
        if with_deg:
            pltpu.make_async_copy(ones_v, deg_sh.at[dstv[b]], dsem.at[b]).wait()

    gather_start(0, 0)
    gather_start(1, 1)
    gather_wait(0)
    scatter_start(0)
    gather_start(2, 2)

    def group(grp, carry):
        for j in range(NBUF):
            g = 1 + grp * NBUF + j
            b = (1 + j) % NBUF
            bm = j
            gather_wait(b)
            scatter_start(b)
            scatter_wait(bm)
            gn = jnp.minimum(g + 2, NCH - 1)
            gather_start(gn, bm)
        return carry

    lax.fori_loop(0, (NCH - 1) // NBUF, group, 0)
    scatter_wait(0)
    gather_wait(1)
    gather_wait(2)
    plsc.subcore_barrier()

    pltpu.sync_copy(acc_sh.at[pl.ds(row0, RPT)], agg_hbm.at[c, pl.ds(row0, RPT)])
    if with_deg:
        pltpu.sync_copy(deg_sh.at[pl.ds(row0, RPT)], deg_hbm.at[c, pl.ds(row0, RPT)])


_ROWB = [pltpu.VMEM((C, D), jnp.float32)] * NBUF
_IDXB = [pltpu.VMEM((C,), jnp.int32)] * (2 * NBUF)

_sc_agg_deg = pl.kernel(
    functools.partial(_sc_body, True),
    out_type=[
        jax.ShapeDtypeStruct((NSC, NP, D), jnp.float32),
        jax.ShapeDtypeStruct((NSC, NP), jnp.float32),
    ],
    mesh=_mesh,
    scratch_types=[
        pltpu.VMEM((NCHP, C), jnp.int32),
        *_ROWB,
        *_IDXB,
        pltpu.VMEM((C,), jnp.float32),
        pltpu.VMEM((C,), jnp.float32),
        pltpu.VMEM_SHARED((NP, D), jnp.float32),
        pltpu.VMEM_SHARED((NP,), jnp.float32),
        pltpu.SemaphoreType.DMA((NBUF,)),
        pltpu.SemaphoreType.DMA((NBUF,)),
        pltpu.SemaphoreType.DMA((NBUF,)),
    ],
)

_sc_agg = pl.kernel(
    functools.partial(_sc_body, False),
    out_type=jax.ShapeDtypeStruct((NSC, NP, D), jnp.float32),
    mesh=_mesh,
    scratch_types=[
        pltpu.VMEM((NCHP, C), jnp.int32),
        *_ROWB,
        *_IDXB,
        pltpu.VMEM_SHARED((NP, D), jnp.float32),
        pltpu.SemaphoreType.DMA((NBUF,)),
        pltpu.SemaphoreType.DMA((NBUF,)),
    ],
)


BM = 2000


def _tc_in_body(x_ref, wn_ref, ws_ref, b_ref, y_ref, z_ref):
    xb = x_ref[...]
    y_ref[...] = jnp.dot(xb, wn_ref[...], preferred_element_type=jnp.float32)
    z_ref[...] = jnp.dot(xb, ws_ref[...], preferred_element_type=jnp.float32) + b_ref[...]


_tc_in = pl.pallas_call(
    _tc_in_body,
    grid=(N // BM,),
    in_specs=[
        pl.BlockSpec((BM, D), lambda i: (i, 0)),
        pl.BlockSpec((D, D), lambda i: (0, 0)),
        pl.BlockSpec((D, D), lambda i: (0, 0)),
        pl.BlockSpec((1, D), lambda i: (0, 0)),
    ],
    out_specs=[
        pl.BlockSpec((BM, D), lambda i: (i, 0)),
        pl.BlockSpec((BM, D), lambda i: (i, 0)),
    ],
    out_shape=[
        jax.ShapeDtypeStruct((N, D), jnp.float32),
        jax.ShapeDtypeStruct((N, D), jnp.float32),
    ],
)


def _h_from_parts(z, aA, aB, dA, dB):
    deg = jnp.maximum(dA + dB, 1.0)
    h = z + (aA + aB) / deg
    return jnp.where(h >= 0, h, 0.01 * h)


_AGG_SPECS = [
    pl.BlockSpec((1, BM, D), lambda i: (0, i, 0)),
    pl.BlockSpec((1, BM, D), lambda i: (1, i, 0)),
    pl.BlockSpec((1, BM, 1), lambda i: (0, i, 0)),
    pl.BlockSpec((1, BM, 1), lambda i: (1, i, 0)),
]


def _tc_mid_body(z_ref, aA_ref, aB_ref, dA_ref, dB_ref, wn_ref, ws_ref, b_ref,
                 y_ref, z2_ref):
    h = _h_from_parts(z_ref[...], aA_ref[0], aB_ref[0], dA_ref[0], dB_ref[0])
    y_ref[...] = jnp.dot(h, wn_ref[...], preferred_element_type=jnp.float32)
    z2_ref[...] = jnp.dot(h, ws_ref[...], preferred_element_type=jnp.float32) + b_ref[...]


_tc_mid = pl.pallas_call(
    _tc_mid_body,
    grid=(N // BM,),
    in_specs=[pl.BlockSpec((BM, D), lambda i: (i, 0))] + _AGG_SPECS + [
        pl.BlockSpec((D, D), lambda i: (0, 0)),
        pl.BlockSpec((D, D), lambda i: (0, 0)),
        pl.BlockSpec((1, D), lambda i: (0, 0)),
    ],
    out_specs=[
        pl.BlockSpec((BM, D), lambda i: (i, 0)),
        pl.BlockSpec((BM, D), lambda i: (i, 0)),
    ],
    out_shape=[
        jax.ShapeDtypeStruct((N, D), jnp.float32),
        jax.ShapeDtypeStruct((N, D), jnp.float32),
    ],
)


def _tc_fin_body(z_ref, aA_ref, aB_ref, dA_ref, dB_ref, mf_ref, w3a_ref,
                 w3b_ref, b3_ref, w4_ref, b4_ref, sum_ref, o_ref):
    h = _h_from_parts(z_ref[...], aA_ref[0], aB_ref[0], dA_ref[0], dB_ref[0])
    part = jnp.sum(h, axis=0, keepdims=True)

    @pl.when(pl.program_id(0) == 0)
    def _init():
        sum_ref[...] = part

    @pl.when(pl.program_id(0) != 0)
    def _acc():
        sum_ref[...] += part

    @pl.when(pl.program_id(0) == pl.num_programs(0) - 1)
    def _head():
        g = sum_ref[...] * (1.0 / N)
        t = (jnp.dot(g, w3a_ref[...], preferred_element_type=jnp.float32)
             + jnp.dot(mf_ref[...], w3b_ref[...], preferred_element_type=jnp.float32)
             + b3_ref[...])
        a = jnp.maximum(t, 0.0)
        o_ref[...] = jnp.dot(a, w4_ref[...], preferred_element_type=jnp.float32) + b4_ref[...]


_tc_fin = pl.pallas_call(
    _tc_fin_body,
    grid=(N // BM,),
    in_specs=[pl.BlockSpec((BM, D), lambda i: (i, 0))] + _AGG_SPECS + [
        pl.BlockSpec((1, MF), lambda i: (0, 0)),
        pl.BlockSpec((D, 64), lambda i: (0, 0)),
        pl.BlockSpec((MF, 64), lambda i: (0, 0)),
        pl.BlockSpec((1, 64), lambda i: (0, 0)),
        pl.BlockSpec((64, 1), lambda i: (0, 0)),
        pl.BlockSpec((1, 1), lambda i: (0, 0)),
    ],
    out_specs=[
        pl.BlockSpec((1, D), lambda i: (0, 0)),
        pl.BlockSpec((1, 1), lambda i: (0, 0)),
    ],
    out_shape=[
        jax.ShapeDtypeStruct((1, D), jnp.float32),
        jax.ShapeDtypeStruct((1, 1), jnp.float32),
    ],
)


def kernel(x, edge_index, manual_features, W1_self, W1_neigh, b1,
           W2_self, W2_neigh, b2, W3, b3, W4, b4):
    packed = (edge_index[0] * 16384 + edge_index[1]).reshape(32, EPS)
    padv = (N + (jnp.arange(32 * EPP, dtype=jnp.int32) % (NP - N))).reshape(32, EPP)
    pk2 = jnp.pad(jnp.concatenate([packed, padv], axis=1).reshape(32, NCH, C),
                  ((0, 0), (0, NCHP - NCH), (0, 0))).reshape(32 * NCHP, C)

    y1, z1 = _tc_in(x, W1_neigh, W1_self, b1.reshape(1, D))
    agg1, deg = _sc_agg_deg(y1, pk2)
    deg3 = deg.reshape(NSC, NP, 1)

    y2, z2 = _tc_mid(z1, agg1, agg1, deg3, deg3,
                     W2_neigh, W2_self, b2.reshape(1, D))
    agg2 = _sc_agg(y2, pk2)

    _, res = _tc_fin(z2, agg2, agg2, deg3, deg3,
                     manual_features.reshape(1, MF),
                     W3[:D], W3[D:], b3.reshape(1, -1), W4, b4.reshape(1, 1))
    return res.reshape((1,))


# speedup vs baseline: 2.0639x; 2.0233x over previous
import functools

import jax
import jax.numpy as jnp
from jax import lax
from jax.experimental import pallas as pl
from jax.experimental.pallas import tpu as pltpu
from jax.experimental.pallas import tpu_sc as plsc

N = 10000
D = 128
E = 320000
MF = 16
NP = 10240
RPT = NP // 16
C = 80
NSC = 2
EPS = E // 32
NCH = EPS // C
NBUF = 2
NCHP = 128

_mesh = plsc.VectorSubcoreMesh(core_axis_name="c", subcore_axis_name="s")


def _sc_body(with_deg, y_hbm, pk_hbm, *rest):
    if with_deg:
        (agg_hbm, deg_hbm, packb, r0, r1, sv0, sv1,
         dv0, dv1, zdeg_v, ones_v,
         acc_sh, deg_sh, gsem, ssem, dsem) = rest
    else:
        (agg_hbm, packb, r0, r1, sv0, sv1,
         dv0, dv1, acc_sh, gsem, ssem) = rest
    rows = (r0, r1)
    srcv = (sv0, sv1)
    dstv = (dv0, dv1)

    c = lax.axis_index("c")
    s = lax.axis_index("s")

    zero16 = jnp.zeros((16,), jnp.float32)
    one16 = jnp.ones((16,), jnp.float32)

    for i in range(C):
        for k in range(D // 16):
            r0[i, pl.ds(k * 16, 16)] = zero16
    if with_deg:
        for i in range(C // 16):
            zdeg_v[pl.ds(i * 16, 16)] = zero16
            ones_v[pl.ds(i * 16, 16)] = one16

    pltpu.sync_copy(pk_hbm.at[pl.ds((c * 16 + s) * NCHP, NCHP)], packb)

    row0 = s * RPT
    for k in range(RPT // C):
        pltpu.sync_copy(r0, acc_sh.at[pl.ds(row0 + k * C, C)])
        if with_deg:
            pltpu.sync_copy(zdeg_v, deg_sh.at[pl.ds(row0 + k * C, C)])
    plsc.subcore_barrier()

    def decode(g, b):
        for k in range(C // 16):
            v16 = packb[g, pl.ds(k * 16, 16)]
            srcv[b][pl.ds(k * 16, 16)] = lax.shift_right_logical(v16, 14)
            dstv[b][pl.ds(k * 16, 16)] = lax.bitwise_and(v16, 16383)

    def gather_start(g, b):
        decode(g, b)
        pltpu.async_copy(y_hbm.at[srcv[b]], rows[b], gsem.at[b])

    def gather_wait(b):
        pltpu.make_async_copy(y_hbm.at[srcv[b]], rows[b], gsem.at[b]).wait()

    def scatter_start(b):
        pltpu.async_copy(rows[b], acc_sh.at[dstv[b]], ssem.at[b], add=True)
        if with_deg:
            pltpu.async_copy(ones_v, deg_sh.at[dstv[b]], dsem.at[b], add=True)

    def scatter_wait(b):
        pltpu.make_async_copy(rows[b], acc_sh.at[dstv[b]], ssem.at[b]).wait()
        if with_deg:
            pltpu.make_async_copy(ones_v, deg_sh.at[dstv[b]], dsem.at[b]).wait()

    for b in range(NBUF):
        gather_start(b, b)

    def group(grp, carry):
        for b in range(NBUF):
            g = grp * NBUF + b
            gather_wait(b)
            scatter_start(b)
            scatter_wait(b)
            gn = jnp.minimum(g + NBUF, NCH - 1)
            gather_start(gn, b)
        return carry

    lax.fori_loop(0, (NCH - 1) // NBUF, group, 0)
    gather_wait(0)
    scatter_start(0)
    scatter_wait(0)
    gather_wait(1)
    plsc.subcore_barrier()

    pltpu.sync_copy(acc_sh.at[pl.ds(row0, RPT)], agg_hbm.at[c, pl.ds(row0, RPT)])
    if with_deg:
        pltpu.sync_copy(deg_sh.at[pl.ds(row0, RPT)], deg_hbm.at[c, pl.ds(row0, RPT)])


_ROWB = [pltpu.VMEM((C, D), jnp.float32)] * NBUF
_IDXB = [pltpu.VMEM((C,), jnp.int32)] * (2 * NBUF)

_sc_agg_deg = pl.kernel(
    functools.partial(_sc_body, True),
    out_type=[
        jax.ShapeDtypeStruct((NSC, NP, D), jnp.float32),
        jax.ShapeDtypeStruct((NSC, NP), jnp.float32),
    ],
    mesh=_mesh,
    scratch_types=[
        pltpu.VMEM((NCHP, C), jnp.int32),
        *_ROWB,
        *_IDXB,
        pltpu.VMEM((C,), jnp.float32),
        pltpu.VMEM((C,), jnp.float32),
        pltpu.VMEM_SHARED((NP, D), jnp.float32),
        pltpu.VMEM_SHARED((NP,), jnp.float32),
        pltpu.SemaphoreType.DMA((NBUF,)),
        pltpu.SemaphoreType.DMA((NBUF,)),
        pltpu.SemaphoreType.DMA((NBUF,)),
    ],
)

_sc_agg = pl.kernel(
    functools.partial(_sc_body, False),
    out_type=jax.ShapeDtypeStruct((NSC, NP, D), jnp.float32),
    mesh=_mesh,
    scratch_types=[
        pltpu.VMEM((NCHP, C), jnp.int32),
        *_ROWB,
        *_IDXB,
        pltpu.VMEM_SHARED((NP, D), jnp.float32),
        pltpu.SemaphoreType.DMA((NBUF,)),
        pltpu.SemaphoreType.DMA((NBUF,)),
    ],
)


BM = 2000


def _tc_in_body(x_ref, wn_ref, ws_ref, b_ref, y_ref, z_ref):
    xb = x_ref[...]
    y_ref[...] = jnp.dot(xb, wn_ref[...], preferred_element_type=jnp.float32)
    z_ref[...] = jnp.dot(xb, ws_ref[...], preferred_element_type=jnp.float32) + b_ref[...]


_tc_in = pl.pallas_call(
    _tc_in_body,
    grid=(N // BM,),
    in_specs=[
        pl.BlockSpec((BM, D), lambda i: (i, 0)),
        pl.BlockSpec((D, D), lambda i: (0, 0)),
        pl.BlockSpec((D, D), lambda i: (0, 0)),
        pl.BlockSpec((1, D), lambda i: (0, 0)),
    ],
    out_specs=[
        pl.BlockSpec((BM, D), lambda i: (i, 0)),
        pl.BlockSpec((BM, D), lambda i: (i, 0)),
    ],
    out_shape=[
        jax.ShapeDtypeStruct((N, D), jnp.float32),
        jax.ShapeDtypeStruct((N, D), jnp.float32),
    ],
)


def _h_from_parts(z, aA, aB, dA, dB):
    deg = jnp.maximum(dA + dB, 1.0)
    h = z + (aA + aB) / deg
    return jnp.where(h >= 0, h, 0.01 * h)


_AGG_SPECS = [
    pl.BlockSpec((1, BM, D), lambda i: (0, i, 0)),
    pl.BlockSpec((1, BM, D), lambda i: (1, i, 0)),
    pl.BlockSpec((1, BM, 1), lambda i: (0, i, 0)),
    pl.BlockSpec((1, BM, 1), lambda i: (1, i, 0)),
]


def _tc_mid_body(z_ref, aA_ref, aB_ref, dA_ref, dB_ref, wn_ref, ws_ref, b_ref,
                 y_ref, z2_ref):
    h = _h_from_parts(z_ref[...], aA_ref[0], aB_ref[0], dA_ref[0], dB_ref[0])
    y_ref[...] = jnp.dot(h, wn_ref[...], preferred_element_type=jnp.float32)
    z2_ref[...] = jnp.dot(h, ws_ref[...], preferred_element_type=jnp.float32) + b_ref[...]


_tc_mid = pl.pallas_call(
    _tc_mid_body,
    grid=(N // BM,),
    in_specs=[pl.BlockSpec((BM, D), lambda i: (i, 0))] + _AGG_SPECS + [
        pl.BlockSpec((D, D), lambda i: (0, 0)),
        pl.BlockSpec((D, D), lambda i: (0, 0)),
        pl.BlockSpec((1, D), lambda i: (0, 0)),
    ],
    out_specs=[
        pl.BlockSpec((BM, D), lambda i: (i, 0)),
        pl.BlockSpec((BM, D), lambda i: (i, 0)),
    ],
    out_shape=[
        jax.ShapeDtypeStruct((N, D), jnp.float32),
        jax.ShapeDtypeStruct((N, D), jnp.float32),
    ],
)


def _tc_fin_body(z_ref, aA_ref, aB_ref, dA_ref, dB_ref, mf_ref, w3a_ref,
                 w3b_ref, b3_ref, w4_ref, b4_ref, sum_ref, o_ref):
    h = _h_from_parts(z_ref[...], aA_ref[0], aB_ref[0], dA_ref[0], dB_ref[0])
    part = jnp.sum(h, axis=0, keepdims=True)

    @pl.when(pl.program_id(0) == 0)
    def _init():
        sum_ref[...] = part

    @pl.when(pl.program_id(0) != 0)
    def _acc():
        sum_ref[...] += part

    @pl.when(pl.program_id(0) == pl.num_programs(0) - 1)
    def _head():
        g = sum_ref[...] * (1.0 / N)
        t = (jnp.dot(g, w3a_ref[...], preferred_element_type=jnp.float32)
             + jnp.dot(mf_ref[...], w3b_ref[...], preferred_element_type=jnp.float32)
             + b3_ref[...])
        a = jnp.maximum(t, 0.0)
        o_ref[...] = jnp.dot(a, w4_ref[...], preferred_element_type=jnp.float32) + b4_ref[...]


_tc_fin = pl.pallas_call(
    _tc_fin_body,
    grid=(N // BM,),
    in_specs=[pl.BlockSpec((BM, D), lambda i: (i, 0))] + _AGG_SPECS + [
        pl.BlockSpec((1, MF), lambda i: (0, 0)),
        pl.BlockSpec((D, 64), lambda i: (0, 0)),
        pl.BlockSpec((MF, 64), lambda i: (0, 0)),
        pl.BlockSpec((1, 64), lambda i: (0, 0)),
        pl.BlockSpec((64, 1), lambda i: (0, 0)),
        pl.BlockSpec((1, 1), lambda i: (0, 0)),
    ],
    out_specs=[
        pl.BlockSpec((1, D), lambda i: (0, 0)),
        pl.BlockSpec((1, 1), lambda i: (0, 0)),
    ],
    out_shape=[
        jax.ShapeDtypeStruct((1, D), jnp.float32),
        jax.ShapeDtypeStruct((1, 1), jnp.float32),
    ],
)


def kernel(x, edge_index, manual_features, W1_self, W1_neigh, b1,
           W2_self, W2_neigh, b2, W3, b3, W4, b4):
    packed = edge_index[0] * 16384 + edge_index[1]
    pk2 = jnp.pad(packed.reshape(32, NCH, C),
                  ((0, 0), (0, NCHP - NCH), (0, 0))).reshape(32 * NCHP, C)

    y1, z1 = _tc_in(x, W1_neigh, W1_self, b1.reshape(1, D))
    agg1, deg = _sc_agg_deg(y1, pk2)
    deg3 = deg.reshape(NSC, NP, 1)

    y2, z2 = _tc_mid(z1, agg1, agg1, deg3, deg3,
                     W2_neigh, W2_self, b2.reshape(1, D))
    agg2 = _sc_agg(y2, pk2)

    _, res = _tc_fin(z2, agg2, agg2, deg3, deg3,
                     manual_features.reshape(1, MF),
                     W3[:D], W3[D:], b3.reshape(1, -1), W4, b4.reshape(1, 1))
    return res.reshape((1,))


# speedup vs baseline: 2.0947x; 1.0149x over previous
import functools

import jax
import jax.numpy as jnp
from jax import lax
from jax.experimental import pallas as pl
from jax.experimental.pallas import tpu as pltpu
from jax.experimental.pallas import tpu_sc as plsc

N = 10000
D = 128
E = 320000
MF = 16
NP = 10240
RPT = NP // 16
C = 80
NSC = 2
EPS = E // 32
NCH = EPS // C
NBUF = 2
NCHP = 128

_mesh = plsc.VectorSubcoreMesh(core_axis_name="c", subcore_axis_name="s")


def _sc_body(with_deg, y_hbm, pk_hbm, *rest):
    if with_deg:
        (agg_hbm, deg_hbm, packb, r0, r1, sv0, sv1,
         dv0, dv1, zdeg_v, ones_v,
         acc_sh, deg_sh, gsem, ssem, dsem) = rest
    else:
        (agg_hbm, packb, r0, r1, sv0, sv1,
         dv0, dv1, acc_sh, gsem, ssem) = rest
    rows = (r0, r1)
    srcv = (sv0, sv1)
    dstv = (dv0, dv1)

    c = lax.axis_index("c")
    s = lax.axis_index("s")

    zero16 = jnp.zeros((16,), jnp.float32)
    one16 = jnp.ones((16,), jnp.float32)

    idx_cp = pltpu.async_copy(
        pk_hbm.at[pl.ds((c * 16 + s) * NCHP, NCHP)], packb, gsem.at[0])

    for i in range(C):
        for k in range(D // 16):
            r0[i, pl.ds(k * 16, 16)] = zero16
    if with_deg:
        for i in range(C // 16):
            zdeg_v[pl.ds(i * 16, 16)] = zero16
            ones_v[pl.ds(i * 16, 16)] = one16

    row0 = s * RPT
    for k in range(RPT // C):
        pltpu.async_copy(r0, acc_sh.at[pl.ds(row0 + k * C, C)], ssem.at[0])
        if with_deg:
            pltpu.async_copy(zdeg_v, deg_sh.at[pl.ds(row0 + k * C, C)], dsem.at[0])
    for k in range(RPT // C):
        pltpu.make_async_copy(r0, acc_sh.at[pl.ds(row0, C)], ssem.at[0]).wait()
        if with_deg:
            pltpu.make_async_copy(zdeg_v, deg_sh.at[pl.ds(row0, C)], dsem.at[0]).wait()
    idx_cp.wait()
    plsc.subcore_barrier()

    def decode(g, b):
        for k in range(C // 16):
            v16 = packb[g, pl.ds(k * 16, 16)]
            srcv[b][pl.ds(k * 16, 16)] = lax.shift_right_logical(v16, 14)
            dstv[b][pl.ds(k * 16, 16)] = lax.bitwise_and(v16, 16383)

    def gather_start(g, b):
        decode(g, b)
        pltpu.async_copy(y_hbm.at[srcv[b]], rows[b], gsem.at[b])

    def gather_wait(b):
        pltpu.make_async_copy(y_hbm.at[srcv[b]], rows[b], gsem.at[b]).wait()

    def scatter_start(b):
        pltpu.async_copy(rows[b], acc_sh.at[dstv[b]], ssem.at[b], add=True)
        if with_deg:
            pltpu.async_copy(ones_v, deg_sh.at[dstv[b]], dsem.at[b], add=True)

    def scatter_wait(b):
        pltpu.make_async_copy(rows[b], acc_sh.at[dstv[b]], ssem.at[b]).wait()
        if with_deg:
            pltpu.make_async_copy(ones_v, deg_sh.at[dstv[b]], dsem.at[b]).wait()

    for b in range(NBUF):
        gather_start(b, b)

    def group(grp, carry):
        for b in range(NBUF):
            g = grp * NBUF + b
            gather_wait(b)
            scatter_start(b)
            scatter_wait(b)
            gn = jnp.minimum(g + NBUF, NCH - 1)
            gather_start(gn, b)
        return carry

    lax.fori_loop(0, (NCH - 1) // NBUF, group, 0)
    gather_wait(0)
    scatter_start(0)
    scatter_wait(0)
    gather_wait(1)
    plsc.subcore_barrier()

    ex1 = pltpu.async_copy(acc_sh.at[pl.ds(row0, RPT)],
                           agg_hbm.at[c, pl.ds(row0, RPT)], gsem.at[0])
    if with_deg:
        ex2 = pltpu.async_copy(deg_sh.at[pl.ds(row0, RPT)],
                               deg_hbm.at[c, pl.ds(row0, RPT)], dsem.at[0])
        ex2.wait()
    ex1.wait()


_ROWB = [pltpu.VMEM((C, D), jnp.float32)] * NBUF
_IDXB = [pltpu.VMEM((C,), jnp.int32)] * (2 * NBUF)

_sc_agg_deg = pl.kernel(
    functools.partial(_sc_body, True),
    out_type=[
        jax.ShapeDtypeStruct((NSC, NP, D), jnp.float32),
        jax.ShapeDtypeStruct((NSC, NP), jnp.float32),
    ],
    mesh=_mesh,
    scratch_types=[
        pltpu.VMEM((NCHP, C), jnp.int32),
        *_ROWB,
        *_IDXB,
        pltpu.VMEM((C,), jnp.float32),
        pltpu.VMEM((C,), jnp.float32),
        pltpu.VMEM_SHARED((NP, D), jnp.float32),
        pltpu.VMEM_SHARED((NP,), jnp.float32),
        pltpu.SemaphoreType.DMA((NBUF,)),
        pltpu.SemaphoreType.DMA((NBUF,)),
        pltpu.SemaphoreType.DMA((NBUF,)),
    ],
)

_sc_agg = pl.kernel(
    functools.partial(_sc_body, False),
    out_type=jax.ShapeDtypeStruct((NSC, NP, D), jnp.float32),
    mesh=_mesh,
    scratch_types=[
        pltpu.VMEM((NCHP, C), jnp.int32),
        *_ROWB,
        *_IDXB,
        pltpu.VMEM_SHARED((NP, D), jnp.float32),
        pltpu.SemaphoreType.DMA((NBUF,)),
        pltpu.SemaphoreType.DMA((NBUF,)),
    ],
)


BM = 2000


def _tc_in_body(x_ref, wn_ref, ws_ref, b_ref, y_ref, z_ref):
    xb = x_ref[...]
    y_ref[...] = jnp.dot(xb, wn_ref[...], preferred_element_type=jnp.float32)
    z_ref[...] = jnp.dot(xb, ws_ref[...], preferred_element_type=jnp.float32) + b_ref[...]


_tc_in = pl.pallas_call(
    _tc_in_body,
    grid=(N // BM,),
    in_specs=[
        pl.BlockSpec((BM, D), lambda i: (i, 0)),
        pl.BlockSpec((D, D), lambda i: (0, 0)),
        pl.BlockSpec((D, D), lambda i: (0, 0)),
        pl.BlockSpec((1, D), lambda i: (0, 0)),
    ],
    out_specs=[
        pl.BlockSpec((BM, D), lambda i: (i, 0)),
        pl.BlockSpec((BM, D), lambda i: (i, 0)),
    ],
    out_shape=[
        jax.ShapeDtypeStruct((N, D), jnp.float32),
        jax.ShapeDtypeStruct((N, D), jnp.float32),
    ],
)


def _h_from_parts(z, aA, aB, dA, dB):
    deg = jnp.maximum(dA + dB, 1.0)
    h = z + (aA + aB) / deg
    return jnp.where(h >= 0, h, 0.01 * h)


_AGG_SPECS = [
    pl.BlockSpec((1, BM, D), lambda i: (0, i, 0)),
    pl.BlockSpec((1, BM, D), lambda i: (1, i, 0)),
    pl.BlockSpec((1, BM, 1), lambda i: (0, i, 0)),
    pl.BlockSpec((1, BM, 1), lambda i: (1, i, 0)),
]


def _tc_mid_body(z_ref, aA_ref, aB_ref, dA_ref, dB_ref, wn_ref, ws_ref, b_ref,
                 y_ref, z2_ref):
    h = _h_from_parts(z_ref[...], aA_ref[0], aB_ref[0], dA_ref[0], dB_ref[0])
    y_ref[...] = jnp.dot(h, wn_ref[...], preferred_element_type=jnp.float32)
    z2_ref[...] = jnp.dot(h, ws_ref[...], preferred_element_type=jnp.float32) + b_ref[...]


_tc_mid = pl.pallas_call(
    _tc_mid_body,
    grid=(N // BM,),
    in_specs=[pl.BlockSpec((BM, D), lambda i: (i, 0))] + _AGG_SPECS + [
        pl.BlockSpec((D, D), lambda i: (0, 0)),
        pl.BlockSpec((D, D), lambda i: (0, 0)),
        pl.BlockSpec((1, D), lambda i: (0, 0)),
    ],
    out_specs=[
        pl.BlockSpec((BM, D), lambda i: (i, 0)),
        pl.BlockSpec((BM, D), lambda i: (i, 0)),
    ],
    out_shape=[
        jax.ShapeDtypeStruct((N, D), jnp.float32),
        jax.ShapeDtypeStruct((N, D), jnp.float32),
    ],
)


def _tc_fin_body(z_ref, aA_ref, aB_ref, dA_ref, dB_ref, mf_ref, w3a_ref,
                 w3b_ref, b3_ref, w4_ref, b4_ref, sum_ref, o_ref):
    h = _h_from_parts(z_ref[...], aA_ref[0], aB_ref[0], dA_ref[0], dB_ref[0])
    part = jnp.sum(h, axis=0, keepdims=True)

    @pl.when(pl.program_id(0) == 0)
    def _init():
        sum_ref[...] = part

    @pl.when(pl.program_id(0) != 0)
    def _acc():
        sum_ref[...] += part

    @pl.when(pl.program_id(0) == pl.num_programs(0) - 1)
    def _head():
        g = sum_ref[...] * (1.0 / N)
        t = (jnp.dot(g, w3a_ref[...], preferred_element_type=jnp.float32)
             + jnp.dot(mf_ref[...], w3b_ref[...], preferred_element_type=jnp.float32)
             + b3_ref[...])
        a = jnp.maximum(t, 0.0)
        o_ref[...] = jnp.dot(a, w4_ref[...], preferred_element_type=jnp.float32) + b4_ref[...]


_tc_fin = pl.pallas_call(
    _tc_fin_body,
    grid=(N // BM,),
    in_specs=[pl.BlockSpec((BM, D), lambda i: (i, 0))] + _AGG_SPECS + [
        pl.BlockSpec((1, MF), lambda i: (0, 0)),
        pl.BlockSpec((D, 64), lambda i: (0, 0)),
        pl.BlockSpec((MF, 64), lambda i: (0, 0)),
        pl.BlockSpec((1, 64), lambda i: (0, 0)),
        pl.BlockSpec((64, 1), lambda i: (0, 0)),
        pl.BlockSpec((1, 1), lambda i: (0, 0)),
    ],
    out_specs=[
        pl.BlockSpec((1, D), lambda i: (0, 0)),
        pl.BlockSpec((1, 1), lambda i: (0, 0)),
    ],
    out_shape=[
        jax.ShapeDtypeStruct((1, D), jnp.float32),
        jax.ShapeDtypeStruct((1, 1), jnp.float32),
    ],
)


def kernel(x, edge_index, manual_features, W1_self, W1_neigh, b1,
           W2_self, W2_neigh, b2, W3, b3, W4, b4):
    packed = edge_index[0] * 16384 + edge_index[1]
    pk2 = jnp.pad(packed.reshape(32, NCH, C),
                  ((0, 0), (0, NCHP - NCH), (0, 0))).reshape(32 * NCHP, C)

    y1, z1 = _tc_in(x, W1_neigh, W1_self, b1.reshape(1, D))
    agg1, deg = _sc_agg_deg(y1, pk2)
    deg3 = deg.reshape(NSC, NP, 1)

    y2, z2 = _tc_mid(z1, agg1, agg1, deg3, deg3,
                     W2_neigh, W2_self, b2.reshape(1, D))
    agg2 = _sc_agg(y2, pk2)

    _, res = _tc_fin(z2, agg2, agg2, deg3, deg3,
                     manual_features.reshape(1, MF),
                     W3[:D], W3[D:], b3.reshape(1, -1), W4, b4.reshape(1, 1))
    return res.reshape((1,))
